# baseline (speedup 1.0000x reference)
import jax
import jax.numpy as jnp
from jax import lax
from jax.experimental import pallas as pl
from jax.experimental.pallas import tpu as pltpu
from jax.experimental.pallas import tpu_sc as plsc

N_NODES = 10000
NODE_DIM = 256
HALF = 128
HID = 512
N_EDGES = 160000
N_TYPES = 5
BLK = 2000
GRID = N_NODES // BLK
EPS = 1e-5

N_SUBCORES = 16
EDGES_PER_TILE = N_EDGES // N_SUBCORES
CHUNK = 80
N_CHUNKS = EDGES_PER_TILE // CHUNK
GROUP = 25
N_GROUPS = N_CHUNKS // GROUP
NODES_PER_TILE = 624
NODES_TAIL = N_NODES - NODES_PER_TILE * N_SUBCORES


def _msg_body(x_hbm, xr_hbm, cidx_hbm, dstr_hbm, out_hbm,
              cidx_v, didx_v, rows_v, acc_sh, sem_r):
    c = lax.axis_index("c")
    s = lax.axis_index("s")
    node_base = c * N_NODES + s * NODES_PER_TILE

    pltpu.sync_copy(x_hbm.at[pl.ds(node_base, NODES_PER_TILE)],
                    acc_sh.at[pl.ds(s * NODES_PER_TILE, NODES_PER_TILE)])

    @pl.when(s == N_SUBCORES - 1)
    def _():
        tail = NODES_PER_TILE * N_SUBCORES
        pltpu.sync_copy(x_hbm.at[pl.ds(c * N_NODES + tail, NODES_TAIL)],
                        acc_sh.at[pl.ds(tail, NODES_TAIL)])

    plsc.subcore_barrier()

    def group_body(g, _):
        pltpu.sync_copy(cidx_hbm.at[c, s, g], cidx_v)
        pltpu.sync_copy(dstr_hbm.at[s, g], didx_v)
        pltpu.async_copy(xr_hbm.at[cidx_v.at[0]], rows_v.at[0], sem_r.at[0])

        def chunk_body(k, _):
            b = k % 2
            bn = (k + 1) % 2

            @pl.when(k + 1 < GROUP)
            def _():
                pltpu.async_copy(xr_hbm.at[cidx_v.at[k + 1]], rows_v.at[bn],
                                 sem_r.at[bn])

            pltpu.make_async_copy(xr_hbm.at[cidx_v.at[k]], rows_v.at[b],
                                  sem_r.at[b]).wait()
            pltpu.sync_copy(rows_v.at[b], acc_sh.at[didx_v.at[k]], add=True)
            return 0

        lax.fori_loop(0, GROUP, chunk_body, 0)
        return 0

    lax.fori_loop(0, N_GROUPS, group_body, 0)
    plsc.subcore_barrier()

    pltpu.sync_copy(acc_sh.at[pl.ds(s * NODES_PER_TILE, NODES_PER_TILE)],
                    out_hbm.at[pl.ds(node_base, NODES_PER_TILE)])

    @pl.when(s == N_SUBCORES - 1)
    def _():
        tail = NODES_PER_TILE * N_SUBCORES
        pltpu.sync_copy(acc_sh.at[pl.ds(tail, NODES_TAIL)],
                        out_hbm.at[pl.ds(c * N_NODES + tail, NODES_TAIL)])


def _msg_call(xflat, xr, cidx, dstr):
    k = pl.kernel(
        _msg_body,
        out_type=jax.ShapeDtypeStruct((2 * N_NODES, HALF), jnp.float32),
        mesh=plsc.VectorSubcoreMesh(core_axis_name="c", subcore_axis_name="s"),
        scratch_types=[
            pltpu.VMEM((GROUP, CHUNK), jnp.int32),
            pltpu.VMEM((GROUP, CHUNK), jnp.int32),
            pltpu.VMEM((2, CHUNK, HALF), jnp.float32),
            pltpu.VMEM_SHARED((N_NODES, HALF), jnp.float32),
            pltpu.SemaphoreType.DMA((2,)),
        ],
    )
    return k(xflat, xr, cidx, dstr)


def _embed_body(at_ref, aemb_ref, etab_ref, out_ref, xr_ref):
    at = at_ref[0, 0]
    onehot = (at[:, None] ==
              lax.broadcasted_iota(jnp.int32, (BLK, 128), 1)).astype(jnp.float32)
    xb = jnp.dot(onehot, aemb_ref[0], preferred_element_type=jnp.float32,
                 precision=lax.Precision.HIGHEST)
    out_ref[...] = xb
    xr = jnp.maximum(xb[:, None, :] + etab_ref[0][None, :, :], 0.0)
    xr_ref[...] = xr.reshape(BLK * N_TYPES, HALF)


def _embed_call(atype2d, aemb_pad, etab2):
    return pl.pallas_call(
        _embed_body,
        grid=(2, GRID),
        in_specs=[
            pl.BlockSpec((1, 1, BLK), lambda j, i: (i, 0, 0)),
            pl.BlockSpec((1, 128, HALF), lambda j, i: (j, 0, 0)),
            pl.BlockSpec((1, N_TYPES, HALF), lambda j, i: (j, 0, 0)),
        ],
        out_specs=[
            pl.BlockSpec((BLK, HALF), lambda j, i: (j * GRID + i, 0)),
            pl.BlockSpec((BLK * N_TYPES, HALF), lambda j, i: (j * GRID + i, 0)),
        ],
        out_shape=[
            jax.ShapeDtypeStruct((2 * N_NODES, HALF), jnp.float32),
            jax.ShapeDtypeStruct((2 * N_NODES * N_TYPES, HALF), jnp.float32),
        ],
    )(atype2d, aemb_pad, etab2)


def _mlp_body(h0lo_ref, h0hi_ref, w1_ref, b1_ref, w2_ref, b2_ref,
              h2_ref, part_ref):
    h1 = (jnp.dot(h0lo_ref[...], w1_ref[0], preferred_element_type=jnp.float32)
          + jnp.dot(h0hi_ref[...], w1_ref[1], preferred_element_type=jnp.float32)
          + b1_ref[...])
    h1 = jnp.maximum(h1, 0.0)
    h2 = jnp.dot(h1, w2_ref[...], preferred_element_type=jnp.float32) + b2_ref[...]
    h2_ref[...] = h2
    mu = jnp.sum(h2, axis=0) * (1.0 / BLK)
    d = h2 - mu
    part_ref[0, 0, :] = mu
    part_ref[0, 1, :] = jnp.sum(d * d, axis=0)


def _mlp_call(h0flat, w1r, b1, w2, b2):
    return pl.pallas_call(
        _mlp_body,
        grid=(GRID,),
        in_specs=[
            pl.BlockSpec((BLK, HALF), lambda i: (i, 0)),
            pl.BlockSpec((BLK, HALF), lambda i: (GRID + i, 0)),
            pl.BlockSpec((2, HALF, HID), lambda i: (0, 0, 0)),
            pl.BlockSpec((1, HID), lambda i: (0, 0)),
            pl.BlockSpec((HID, NODE_DIM), lambda i: (0, 0)),
            pl.BlockSpec((1, NODE_DIM), lambda i: (0, 0)),
        ],
        out_specs=[
            pl.BlockSpec((BLK, NODE_DIM), lambda i: (i, 0)),
            pl.BlockSpec((1, 2, NODE_DIM), lambda i: (i, 0, 0)),
        ],
        out_shape=[
            jax.ShapeDtypeStruct((N_NODES, NODE_DIM), jnp.float32),
            jax.ShapeDtypeStruct((GRID, 2, NODE_DIM), jnp.float32),
        ],
    )(h0flat, h0flat, w1r, b1.reshape(1, HID), w2, b2.reshape(1, NODE_DIM))


def _bn_mid_body(h2_ref, part_ref, res_ref, gamma_ref, beta_ref, etab_ref,
                 out_ref, xr_ref):
    mus = part_ref[:, 0, :]
    m2s = part_ref[:, 1, :]
    mean = jnp.sum(mus, axis=0) * (1.0 / GRID)
    dm = mus - mean
    var = (jnp.sum(m2s, axis=0) + BLK * jnp.sum(dm * dm, axis=0)) * (1.0 / N_NODES)
    rstd = lax.rsqrt(var + EPS)
    h = (h2_ref[...] - mean) * (rstd * gamma_ref[0]) + beta_ref[0]
    x_new = jnp.maximum(h, 0.0) + res_ref[...]
    out_ref[...] = x_new
    xr = jnp.maximum(x_new[:, None, :] + etab_ref[0][None, :, :], 0.0)
    xr_ref[...] = xr.reshape(BLK * N_TYPES, HALF)


def _bn_final_body(h2_ref, part_ref, res_ref, gamma_ref, beta_ref, out_ref):
    mus = part_ref[:, 0, :]
    m2s = part_ref[:, 1, :]
    mean = jnp.sum(mus, axis=0) * (1.0 / GRID)
    dm = mus - mean
    var = (jnp.sum(m2s, axis=0) + BLK * jnp.sum(dm * dm, axis=0)) * (1.0 / N_NODES)
    rstd = lax.rsqrt(var + EPS)
    h = (h2_ref[...] - mean) * (rstd * gamma_ref[0]) + beta_ref[0]
    out_ref[...] = jnp.maximum(h, 0.0) + res_ref[...]


def _bn_call(h2, part, xflat, gamma, beta, etab2, final):
    in_specs = [
        pl.BlockSpec((BLK, HALF), lambda j, i: (i, j)),
        pl.BlockSpec((GRID, 2, HALF), lambda j, i: (0, 0, j)),
        pl.BlockSpec((BLK, HALF), lambda j, i: (j * GRID + i, 0)),
        pl.BlockSpec((1, HALF), lambda j, i: (0, j)),
        pl.BlockSpec((1, HALF), lambda j, i: (0, j)),
    ]
    g2 = gamma.reshape(1, NODE_DIM)
    b2_ = beta.reshape(1, NODE_DIM)
    if final:
        return pl.pallas_call(
            _bn_final_body,
            grid=(2, GRID),
            in_specs=in_specs,
            out_specs=pl.BlockSpec((BLK, HALF), lambda j, i: (i, j)),
            out_shape=jax.ShapeDtypeStruct((N_NODES, NODE_DIM), jnp.float32),
        )(h2, part, xflat, g2, b2_)
    return pl.pallas_call(
        _bn_mid_body,
        grid=(2, GRID),
        in_specs=in_specs + [pl.BlockSpec((1, N_TYPES, HALF), lambda j, i: (j, 0, 0))],
        out_specs=[
            pl.BlockSpec((BLK, HALF), lambda j, i: (j * GRID + i, 0)),
            pl.BlockSpec((BLK * N_TYPES, HALF), lambda j, i: (j * GRID + i, 0)),
        ],
        out_shape=[
            jax.ShapeDtypeStruct((2 * N_NODES, HALF), jnp.float32),
            jax.ShapeDtypeStruct((2 * N_NODES * N_TYPES, HALF), jnp.float32),
        ],
    )(h2, part, xflat, g2, b2_, etab2)


def kernel(atom_type, edge_index, edge_type, atom_emb, edge_emb,
           W1, b1, W2, b2, gamma, beta):
    num_layers = W1.shape[0]
    src = edge_index[0].astype(jnp.int32)
    dst = edge_index[1].astype(jnp.int32)
    et = edge_type.astype(jnp.int32)

    ci = src * N_TYPES + et
    cidx = jnp.stack([ci, ci + N_NODES * N_TYPES]).reshape(
        2, N_SUBCORES, N_GROUPS, GROUP, CHUNK)
    dstr = dst.reshape(N_SUBCORES, N_GROUPS, GROUP, CHUNK)

    etab2 = edge_emb.reshape(N_TYPES, 2, HALF).transpose(1, 0, 2)
    aemb_pad = jnp.zeros((2, 128, HALF), jnp.float32)
    aemb_pad = aemb_pad.at[:, :119, :].set(
        atom_emb.reshape(119, 2, HALF).transpose(1, 0, 2))
    atype2d = atom_type.astype(jnp.int32).reshape(GRID, 1, BLK)

    xflat, xr = _embed_call(atype2d, aemb_pad, etab2)

    out = None
    for l in range(num_layers):
        h0flat = _msg_call(xflat, xr, cidx, dstr)
        w1r = W1[l].reshape(2, HALF, HID)
        h2, part = _mlp_call(h0flat, w1r, b1[l], W2[l], b2[l])
        if l == num_layers - 1:
            out = _bn_call(h2, part, xflat, gamma[l], beta[l], etab2, True)
        else:
            xflat, xr = _bn_call(h2, part, xflat, gamma[l], beta[l], etab2, False)
    return out


# speedup vs baseline: 1.0303x; 1.0303x over previous
import jax
import jax.numpy as jnp
from jax import lax
from jax.experimental import pallas as pl
from jax.experimental.pallas import tpu as pltpu
from jax.experimental.pallas import tpu_sc as plsc

N_NODES = 10000
NODE_DIM = 256
HALF = 128
HID = 512
N_EDGES = 160000
N_TYPES = 5
BLK = 2000
GRID = N_NODES // BLK
EPS = 1e-5

N_SUBCORES = 16
EDGES_PER_TILE = N_EDGES // N_SUBCORES
CHUNK = 100
N_CHUNKS = EDGES_PER_TILE // CHUNK
GROUP = 20
N_GROUPS = N_CHUNKS // GROUP
NODES_PER_TILE = 624
NODES_TAIL = N_NODES - NODES_PER_TILE * N_SUBCORES


def _msg_body(x_hbm, xr_hbm, cidx_hbm, dstr_hbm, out_hbm,
              cidx_v, didx_v, rows_v, acc_sh, sem_r):
    c = lax.axis_index("c")
    s = lax.axis_index("s")
    node_base = c * N_NODES + s * NODES_PER_TILE

    pltpu.sync_copy(x_hbm.at[pl.ds(node_base, NODES_PER_TILE)],
                    acc_sh.at[pl.ds(s * NODES_PER_TILE, NODES_PER_TILE)])

    @pl.when(s == N_SUBCORES - 1)
    def _():
        tail = NODES_PER_TILE * N_SUBCORES
        pltpu.sync_copy(x_hbm.at[pl.ds(c * N_NODES + tail, NODES_TAIL)],
                        acc_sh.at[pl.ds(tail, NODES_TAIL)])

    plsc.subcore_barrier()

    def group_body(g, _):
        pltpu.sync_copy(cidx_hbm.at[c, s, g], cidx_v)
        pltpu.sync_copy(dstr_hbm.at[s, g], didx_v)
        pltpu.async_copy(xr_hbm.at[cidx_v.at[0]], rows_v.at[0], sem_r.at[0])

        def chunk_body(k, _):
            b = k % 2
            bn = (k + 1) % 2

            @pl.when(k + 1 < GROUP)
            def _():
                pltpu.async_copy(xr_hbm.at[cidx_v.at[k + 1]], rows_v.at[bn],
                                 sem_r.at[bn])

            pltpu.make_async_copy(xr_hbm.at[cidx_v.at[k]], rows_v.at[b],
                                  sem_r.at[b]).wait()
            pltpu.sync_copy(rows_v.at[b], acc_sh.at[didx_v.at[k]], add=True)
            return 0

        lax.fori_loop(0, GROUP, chunk_body, 0)
        return 0

    lax.fori_loop(0, N_GROUPS, group_body, 0)
    plsc.subcore_barrier()

    pltpu.sync_copy(acc_sh.at[pl.ds(s * NODES_PER_TILE, NODES_PER_TILE)],
                    out_hbm.at[pl.ds(node_base, NODES_PER_TILE)])

    @pl.when(s == N_SUBCORES - 1)
    def _():
        tail = NODES_PER_TILE * N_SUBCORES
        pltpu.sync_copy(acc_sh.at[pl.ds(tail, NODES_TAIL)],
                        out_hbm.at[pl.ds(c * N_NODES + tail, NODES_TAIL)])


def _msg_call(xflat, xr, cidx, dstr):
    k = pl.kernel(
        _msg_body,
        out_type=jax.ShapeDtypeStruct((2 * N_NODES, HALF), jnp.float32),
        mesh=plsc.VectorSubcoreMesh(core_axis_name="c", subcore_axis_name="s"),
        scratch_types=[
            pltpu.VMEM((GROUP, CHUNK), jnp.int32),
            pltpu.VMEM((GROUP, CHUNK), jnp.int32),
            pltpu.VMEM((2, CHUNK, HALF), jnp.float32),
            pltpu.VMEM_SHARED((N_NODES, HALF), jnp.float32),
            pltpu.SemaphoreType.DMA((2,)),
        ],
    )
    return k(xflat, xr, cidx, dstr)


def _embed_body(at_ref, aemb_ref, etab_ref, out_ref, xr_ref):
    at = at_ref[0, 0]
    onehot = (at[:, None] ==
              lax.broadcasted_iota(jnp.int32, (BLK, 128), 1)).astype(jnp.float32)
    xb = jnp.dot(onehot, aemb_ref[0], preferred_element_type=jnp.float32,
                 precision=lax.Precision.HIGHEST)
    out_ref[...] = xb
    xr = jnp.maximum(xb[:, None, :] + etab_ref[0][None, :, :], 0.0)
    xr_ref[...] = xr.reshape(BLK * N_TYPES, HALF)


def _embed_call(atype2d, aemb_pad, etab2):
    return pl.pallas_call(
        _embed_body,
        grid=(2, GRID),
        in_specs=[
            pl.BlockSpec((1, 1, BLK), lambda j, i: (i, 0, 0)),
            pl.BlockSpec((1, 128, HALF), lambda j, i: (j, 0, 0)),
            pl.BlockSpec((1, N_TYPES, HALF), lambda j, i: (j, 0, 0)),
        ],
        out_specs=[
            pl.BlockSpec((BLK, HALF), lambda j, i: (j * GRID + i, 0)),
            pl.BlockSpec((BLK * N_TYPES, HALF), lambda j, i: (j * GRID + i, 0)),
        ],
        out_shape=[
            jax.ShapeDtypeStruct((2 * N_NODES, HALF), jnp.float32),
            jax.ShapeDtypeStruct((2 * N_NODES * N_TYPES, HALF), jnp.float32),
        ],
    )(atype2d, aemb_pad, etab2)


def _mlp_body(h0lo_ref, h0hi_ref, w1_ref, b1_ref, w2_ref, b2_ref,
              h2_ref, part_ref):
    h1 = (jnp.dot(h0lo_ref[...], w1_ref[0], preferred_element_type=jnp.float32)
          + jnp.dot(h0hi_ref[...], w1_ref[1], preferred_element_type=jnp.float32)
          + b1_ref[...])
    h1 = jnp.maximum(h1, 0.0)
    h2 = jnp.dot(h1, w2_ref[...], preferred_element_type=jnp.float32) + b2_ref[...]
    h2_ref[...] = h2
    mu = jnp.sum(h2, axis=0) * (1.0 / BLK)
    d = h2 - mu
    part_ref[0, 0, :] = mu
    part_ref[0, 1, :] = jnp.sum(d * d, axis=0)


def _mlp_call(h0flat, w1r, b1, w2, b2):
    return pl.pallas_call(
        _mlp_body,
        grid=(GRID,),
        in_specs=[
            pl.BlockSpec((BLK, HALF), lambda i: (i, 0)),
            pl.BlockSpec((BLK, HALF), lambda i: (GRID + i, 0)),
            pl.BlockSpec((2, HALF, HID), lambda i: (0, 0, 0)),
            pl.BlockSpec((1, HID), lambda i: (0, 0)),
            pl.BlockSpec((HID, NODE_DIM), lambda i: (0, 0)),
            pl.BlockSpec((1, NODE_DIM), lambda i: (0, 0)),
        ],
        out_specs=[
            pl.BlockSpec((BLK, NODE_DIM), lambda i: (i, 0)),
            pl.BlockSpec((1, 2, NODE_DIM), lambda i: (i, 0, 0)),
        ],
        out_shape=[
            jax.ShapeDtypeStruct((N_NODES, NODE_DIM), jnp.float32),
            jax.ShapeDtypeStruct((GRID, 2, NODE_DIM), jnp.float32),
        ],
    )(h0flat, h0flat, w1r, b1.reshape(1, HID), w2, b2.reshape(1, NODE_DIM))


def _bn_mid_body(h2_ref, part_ref, res_ref, gamma_ref, beta_ref, etab_ref,
                 out_ref, xr_ref):
    mus = part_ref[:, 0, :]
    m2s = part_ref[:, 1, :]
    mean = jnp.sum(mus, axis=0) * (1.0 / GRID)
    dm = mus - mean
    var = (jnp.sum(m2s, axis=0) + BLK * jnp.sum(dm * dm, axis=0)) * (1.0 / N_NODES)
    rstd = lax.rsqrt(var + EPS)
    h = (h2_ref[...] - mean) * (rstd * gamma_ref[0]) + beta_ref[0]
    x_new = jnp.maximum(h, 0.0) + res_ref[...]
    out_ref[...] = x_new
    xr = jnp.maximum(x_new[:, None, :] + etab_ref[0][None, :, :], 0.0)
    xr_ref[...] = xr.reshape(BLK * N_TYPES, HALF)


def _bn_final_body(h2_ref, part_ref, res_ref, gamma_ref, beta_ref, out_ref):
    mus = part_ref[:, 0, :]
    m2s = part_ref[:, 1, :]
    mean = jnp.sum(mus, axis=0) * (1.0 / GRID)
    dm = mus - mean
    var = (jnp.sum(m2s, axis=0) + BLK * jnp.sum(dm * dm, axis=0)) * (1.0 / N_NODES)
    rstd = lax.rsqrt(var + EPS)
    h = (h2_ref[...] - mean) * (rstd * gamma_ref[0]) + beta_ref[0]
    out_ref[...] = jnp.maximum(h, 0.0) + res_ref[...]


def _bn_call(h2, part, xflat, gamma, beta, etab2, final):
    in_specs = [
        pl.BlockSpec((BLK, HALF), lambda j, i: (i, j)),
        pl.BlockSpec((GRID, 2, HALF), lambda j, i: (0, 0, j)),
        pl.BlockSpec((BLK, HALF), lambda j, i: (j * GRID + i, 0)),
        pl.BlockSpec((1, HALF), lambda j, i: (0, j)),
        pl.BlockSpec((1, HALF), lambda j, i: (0, j)),
    ]
    g2 = gamma.reshape(1, NODE_DIM)
    b2_ = beta.reshape(1, NODE_DIM)
    if final:
        return pl.pallas_call(
            _bn_final_body,
            grid=(2, GRID),
            in_specs=in_specs,
            out_specs=pl.BlockSpec((BLK, HALF), lambda j, i: (i, j)),
            out_shape=jax.ShapeDtypeStruct((N_NODES, NODE_DIM), jnp.float32),
        )(h2, part, xflat, g2, b2_)
    return pl.pallas_call(
        _bn_mid_body,
        grid=(2, GRID),
        in_specs=in_specs + [pl.BlockSpec((1, N_TYPES, HALF), lambda j, i: (j, 0, 0))],
        out_specs=[
            pl.BlockSpec((BLK, HALF), lambda j, i: (j * GRID + i, 0)),
            pl.BlockSpec((BLK * N_TYPES, HALF), lambda j, i: (j * GRID + i, 0)),
        ],
        out_shape=[
            jax.ShapeDtypeStruct((2 * N_NODES, HALF), jnp.float32),
            jax.ShapeDtypeStruct((2 * N_NODES * N_TYPES, HALF), jnp.float32),
        ],
    )(h2, part, xflat, g2, b2_, etab2)


def kernel(atom_type, edge_index, edge_type, atom_emb, edge_emb,
           W1, b1, W2, b2, gamma, beta):
    num_layers = W1.shape[0]
    src = edge_index[0].astype(jnp.int32)
    dst = edge_index[1].astype(jnp.int32)
    et = edge_type.astype(jnp.int32)

    ci = src * N_TYPES + et
    cidx = jnp.stack([ci, ci + N_NODES * N_TYPES]).reshape(
        2, N_SUBCORES, N_GROUPS, GROUP, CHUNK)
    dstr = dst.reshape(N_SUBCORES, N_GROUPS, GROUP, CHUNK)

    etab2 = edge_emb.reshape(N_TYPES, 2, HALF).transpose(1, 0, 2)
    aemb_pad = jnp.zeros((2, 128, HALF), jnp.float32)
    aemb_pad = aemb_pad.at[:, :119, :].set(
        atom_emb.reshape(119, 2, HALF).transpose(1, 0, 2))
    atype2d = atom_type.astype(jnp.int32).reshape(GRID, 1, BLK)

    xflat, xr = _embed_call(atype2d, aemb_pad, etab2)

    out = None
    for l in range(num_layers):
        h0flat = _msg_call(xflat, xr, cidx, dstr)
        w1r = W1[l].reshape(2, HALF, HID)
        h2, part = _mlp_call(h0flat, w1r, b1[l], W2[l], b2[l])
        if l == num_layers - 1:
            out = _bn_call(h2, part, xflat, gamma[l], beta[l], etab2, True)
        else:
            xflat, xr = _bn_call(h2, part, xflat, gamma[l], beta[l], etab2, False)
    return out
